# baseline (speedup 1.0000x reference)
import functools

import jax
import jax.numpy as jnp
from jax import lax
from jax.experimental import pallas as pl
from jax.experimental.pallas import tpu as pltpu
from jax.experimental.pallas import tpu_sc as plsc

CTX = 1000
B, S = 1024, 50
ROWS = B * S
NW = 32
PAIRW = 2 * CTX
NPAIR = ROWS // 2
PPW = NPAIR // NW
CP = 16
NCHUNK = PPW // CP
CHUNK = 2 * CP
L = 16
NBUF = 2


def _body(seq_hbm, out_hbm, seq_v, buf0, buf1, sem0, sem1):
    cid = lax.axis_index("c")
    sid = lax.axis_index("s")
    wid = sid * 2 + cid

    pltpu.sync_copy(seq_hbm, seq_v)

    zero16 = jnp.zeros((L,), jnp.float32)
    one16 = jnp.full((L,), 1.0, jnp.float32)
    iota16 = lax.iota(jnp.int32, L)

    bufs = (buf0, buf1)
    sems = (sem0, sem1)

    def _zero_body(i, carry):
        base = i * L
        for p in range(CP):
            for bz in bufs:
                bz[p, pl.ds(base, L)] = zero16
        return carry
    lax.fori_loop(0, PAIRW // L, _zero_body, 0)

    def _buf_idx(c, o):
        r = iota16 + o
        pair_local = c * CP + (r >> 1)
        gpair = wid + pair_local * NW
        grow = 2 * gpair + (r & 1)
        vals = plsc.load_gather(seq_v, [grow])
        rows = r >> 1
        cols = (r & 1) * CTX + vals
        return rows, cols

    handles = [None] * NBUF
    pending = [None] * NBUF
    for c in range(NCHUNK):
        bsel = c % NBUF
        buf = bufs[bsel]
        if handles[bsel] is not None:
            handles[bsel].wait()
            pc = pending[bsel]
            for o in range(0, CHUNK, L):
                rows, cols = _buf_idx(pc, o)
                plsc.store_scatter(buf, [rows, cols], zero16)
        for o in range(0, CHUNK, L):
            rows, cols = _buf_idx(c, o)
            plsc.store_scatter(buf, [rows, cols], one16)
        pairs = wid + (c * CP + iota16) * NW
        handles[bsel] = pltpu.async_copy(buf, out_hbm.at[pairs], sems[bsel])
        pending[bsel] = c
    for h in handles:
        h.wait()


@jax.jit
def _onehot_sc(seq_flat):
    kern = functools.partial(
        pl.kernel,
        mesh=plsc.VectorSubcoreMesh(core_axis_name="c", subcore_axis_name="s"),
        out_type=jax.ShapeDtypeStruct((NPAIR, PAIRW), jnp.float32),
        scratch_types=[
            pltpu.VMEM((ROWS,), jnp.int32),
            pltpu.VMEM((CP, PAIRW), jnp.float32),
            pltpu.VMEM((CP, PAIRW), jnp.float32),
            pltpu.SemaphoreType.DMA,
            pltpu.SemaphoreType.DMA,
        ],
        compiler_params=pltpu.CompilerParams(
            needs_layout_passes=False, use_tc_tiling_on_sc=False),
    )(_body)
    return kern(seq_flat)


def kernel(sequence):
    seq_flat = sequence.reshape(ROWS).astype(jnp.int32)
    out = _onehot_sc(seq_flat)
    return out.reshape(B, S, CTX)


# speedup vs baseline: 1.8959x; 1.8959x over previous
import functools

import jax
import jax.numpy as jnp
from jax import lax
from jax.experimental import pallas as pl
from jax.experimental.pallas import tpu as pltpu
from jax.experimental.pallas import tpu_sc as plsc

CTX = 1000
B, S = 1024, 50
ROWS = B * S
NW = 32
BPW = B // NW
L = 16
NBUF = 2


def _body(seq_hbm, out_hbm, idx_v, buf0, buf1, sem0, sem1):
    cid = lax.axis_index("c")
    sid = lax.axis_index("s")
    wid = sid * 2 + cid
    b0 = wid * BPW
    row0 = b0 * S

    pltpu.sync_copy(seq_hbm.at[pl.ds(row0, BPW * S)],
                    idx_v.at[pl.ds(0, BPW * S)])
    idx_v[pl.ds(BPW * S, L)] = jnp.zeros((L,), jnp.int32)

    zero16 = jnp.zeros((L,), jnp.float32)
    one16 = jnp.full((L,), 1.0, jnp.float32)
    iota16 = lax.iota(jnp.int32, L)
    mask2 = iota16 < (S - 3 * L)

    bufs = (buf0, buf1)
    sems = (sem0, sem1)

    col_starts = list(range(0, CTX - L, L)) + [CTX - L]
    def _zero_body(s, carry):
        for bz in bufs:
            for base in col_starts:
                bz[0, s, pl.ds(base, L)] = zero16
        return carry
    lax.fori_loop(0, S, _zero_body, 0)

    def _groups(c):
        out = []
        for o in range(0, S, L):
            n = min(L, S - o)
            vals = idx_v[pl.ds(c * S + o, L)]
            out.append((iota16 + o, vals, None if n == L else mask2))
        return out

    handles = [None] * NBUF
    pending = [None] * NBUF
    for c in range(BPW):
        bsel = c % NBUF
        buf = bufs[bsel]
        if handles[bsel] is not None:
            handles[bsel].wait()
            pc = pending[bsel]
            for rows, cols, m in _groups(pc):
                plsc.store_scatter(buf, [rows - rows, rows, cols], zero16, mask=m)
        for rows, cols, m in _groups(c):
            plsc.store_scatter(buf, [rows - rows, rows, cols], one16, mask=m)
        handles[bsel] = pltpu.async_copy(buf, out_hbm.at[pl.ds(b0 + c, 1)], sems[bsel])
        pending[bsel] = c
    for h in handles:
        h.wait()


@jax.jit
def _onehot_sc(seq_flat):
    kern = functools.partial(
        pl.kernel,
        mesh=plsc.VectorSubcoreMesh(core_axis_name="c", subcore_axis_name="s"),
        out_type=jax.ShapeDtypeStruct((B, S, CTX), jnp.float32),
        scratch_types=[
            pltpu.VMEM((BPW * S + L,), jnp.int32),
            pltpu.VMEM((1, S, CTX), jnp.float32),
            pltpu.VMEM((1, S, CTX), jnp.float32),
            pltpu.SemaphoreType.DMA,
            pltpu.SemaphoreType.DMA,
        ],
        compiler_params=pltpu.CompilerParams(needs_layout_passes=False),
    )(_body)
    return kern(seq_flat)


def kernel(sequence):
    seq_flat = sequence.reshape(ROWS).astype(jnp.int32)
    return _onehot_sc(seq_flat)


# speedup vs baseline: 1.9089x; 1.0069x over previous
import functools

import jax
import jax.numpy as jnp
from jax import lax
from jax.experimental import pallas as pl
from jax.experimental.pallas import tpu as pltpu
from jax.experimental.pallas import tpu_sc as plsc

CTX = 1000
B, S = 1024, 50
ROWS = B * S
NW = 32
BPW = B // NW
L = 16
NBUF = 2


def _body(seq_hbm, out_hbm, idx_v, buf0, buf1, sem0, sem1):
    cid = lax.axis_index("c")
    sid = lax.axis_index("s")
    wid = sid * 2 + cid
    b0 = wid * BPW
    row0 = b0 * S

    pltpu.sync_copy(seq_hbm.at[pl.ds(row0, BPW * S)],
                    idx_v.at[pl.ds(0, BPW * S)])
    idx_v[pl.ds(BPW * S, L)] = jnp.zeros((L,), jnp.int32)

    zero16 = jnp.zeros((L,), jnp.float32)
    one16 = jnp.full((L,), 1.0, jnp.float32)
    iota16 = lax.iota(jnp.int32, L)
    mask2 = iota16 < (S - 3 * L)

    bufs = (buf0, buf1)
    sems = (sem0, sem1)

    col_starts = list(range(0, CTX - L, L)) + [CTX - L]
    def _zero_body(s, carry):
        for bz in bufs:
            for base in col_starts:
                bz[0, s, pl.ds(base, L)] = zero16
        return carry
    lax.fori_loop(0, S, _zero_body, 0)

    zrow = iota16 - iota16

    def _patch(buf, c, val16):
        for o in range(0, S, L):
            n = min(L, S - o)
            vals = idx_v[pl.ds(c * S + o, L)]
            m = None if n == L else mask2
            plsc.store_scatter(buf, [zrow, iota16 + o, vals], val16, mask=m)

    def _fire(buf, c, sem):
        return pltpu.async_copy(buf, out_hbm.at[pl.ds(b0 + c, 1)], sem)

    _patch(buf0, 0, one16)
    _fire(buf0, 0, sem0)
    _patch(buf1, 1, one16)
    _fire(buf1, 1, sem1)

    def _loop_body(i, carry):
        for b, (buf, sem) in enumerate(zip(bufs, sems)):
            c = 2 * i + 2 + b
            pltpu.make_async_copy(buf, out_hbm.at[pl.ds(b0 + c, 1)], sem).wait()
            _patch(buf, c - 2, zero16)
            _patch(buf, c, one16)
            _fire(buf, c, sem)
        return carry
    lax.fori_loop(0, (BPW - NBUF) // NBUF, _loop_body, 0)

    for b, (buf, sem) in enumerate(zip(bufs, sems)):
        pltpu.make_async_copy(buf, out_hbm.at[pl.ds(b0 + b, 1)], sem).wait()


@jax.jit
def _onehot_sc(seq_flat):
    kern = functools.partial(
        pl.kernel,
        mesh=plsc.VectorSubcoreMesh(core_axis_name="c", subcore_axis_name="s"),
        out_type=jax.ShapeDtypeStruct((B, S, CTX), jnp.float32),
        scratch_types=[
            pltpu.VMEM((BPW * S + L,), jnp.int32),
            pltpu.VMEM((1, S, CTX), jnp.float32),
            pltpu.VMEM((1, S, CTX), jnp.float32),
            pltpu.SemaphoreType.DMA,
            pltpu.SemaphoreType.DMA,
        ],
        compiler_params=pltpu.CompilerParams(needs_layout_passes=False),
    )(_body)
    return kern(seq_flat)


def kernel(sequence):
    seq_flat = sequence.reshape(ROWS).astype(jnp.int32)
    return _onehot_sc(seq_flat)
